# Initial kernel scaffold; baseline (speedup 1.0000x reference)
import functools

import jax
import jax.numpy as jnp
from jax import lax
from jax.experimental import pallas as pl
from jax.experimental.pallas import tpu as pltpu
from jax.experimental.pallas import tpu_sc as plsc

N = 10000
E = 640000
D = 128
H = 64
G = 64

CHUNK = 128
NUM_CHUNKS = E // CHUNK
NC = 2
NS = 16
NW = NC * NS
CHUNKS_PER_TILE = (NUM_CHUNKS + NW - 1) // NW
ROWS_PER_SUB = N // NS


def _make_seg_sum(width):
  mesh = plsc.VectorSubcoreMesh(core_axis_name="c", subcore_axis_name="s")

  @functools.partial(
      pl.kernel,
      out_type=jax.ShapeDtypeStruct((2 * N, width), jnp.float32),
      mesh=mesh,
      scratch_types=[
          pltpu.VMEM((CHUNK,), jnp.int32),
          pltpu.VMEM((CHUNK,), jnp.int32),
          pltpu.VMEM((CHUNK, width), jnp.float32),
          pltpu.VMEM_SHARED((N, width), jnp.float32),
          pltpu.SemaphoreType.DMA,
      ],
  )
  def seg_sum(feat_hbm, ei_hbm, zeros_hbm, out_hbm,
              src_v, dst_v, rows_v, acc_sh, sem):
    c = lax.axis_index("c")
    s = lax.axis_index("s")
    wid = s * NC + c

    pltpu.sync_copy(zeros_hbm.at[pl.ds(s * ROWS_PER_SUB, ROWS_PER_SUB)],
                    acc_sh.at[pl.ds(s * ROWS_PER_SUB, ROWS_PER_SUB)])
    plsc.subcore_barrier()

    def body(k, carry):
      chunk = k * NW + wid

      @pl.when(chunk < NUM_CHUNKS)
      def _():
        base = chunk * CHUNK
        pltpu.sync_copy(ei_hbm.at[0, pl.ds(base, CHUNK)], src_v)
        pltpu.sync_copy(ei_hbm.at[1, pl.ds(base, CHUNK)], dst_v)
        pltpu.async_copy(feat_hbm.at[src_v], rows_v, sem).wait()
        pltpu.sync_copy(rows_v, acc_sh.at[dst_v], add=True)

      return carry

    lax.fori_loop(0, CHUNKS_PER_TILE, body, 0)
    plsc.subcore_barrier()

    pltpu.sync_copy(acc_sh.at[pl.ds(s * ROWS_PER_SUB, ROWS_PER_SUB)],
                    out_hbm.at[pl.ds(c * N + s * ROWS_PER_SUB, ROWS_PER_SUB)])

  return seg_sum


_seg_sum_128 = _make_seg_sum(D)
_seg_sum_64 = _make_seg_sum(H)


def _bn(a, gamma, beta, eps=1e-5):
  m = jnp.mean(a, axis=0, keepdims=True)
  v = jnp.mean((a - m) ** 2, axis=0, keepdims=True)
  return gamma * (a - m) / jnp.sqrt(v + eps) + beta


def _mlp(h, p):
  a = jnp.dot(h, p["W1"][...], preferred_element_type=jnp.float32) + p["b1"][...]
  a = jnp.maximum(_bn(a, p["g1"][...], p["be1"][...]), 0.0)
  a = jnp.dot(a, p["W2"][...], preferred_element_type=jnp.float32) + p["b2"][...]
  return jnp.maximum(_bn(a, p["g2"][...], p["be2"][...]), 0.0)


def _gin_body(has_residual, x_ref, p_ref, params_refs, o_ref):
  x = x_ref[...]
  h = x + p_ref[0] + p_ref[1]
  out = _mlp(h, params_refs)
  o_ref[...] = x + out if has_residual else out


def _gin_call(x, p, mlp_params, has_residual):
  body = functools.partial(_gin_body, has_residual)
  return pl.pallas_call(
      body,
      out_shape=jax.ShapeDtypeStruct((N, H), jnp.float32),
  )(x, p, mlp_params)


def _head(g, p):
  t = jnp.maximum(
      jnp.dot(g, p["W1"][...], preferred_element_type=jnp.float32) + p["b1"][...], 0.0)
  z = jnp.dot(t, p["W2"][...], preferred_element_type=jnp.float32) + p["b2"][...]
  n = jnp.sqrt(jnp.sum(z * z, axis=1, keepdims=True))
  return z / jnp.maximum(n, 1e-12)


def _final_body(x_ref, p_ref, conv3_refs, batch_ref, hc_refs, hs_refs, hg_refs,
                ht_refs, oc_ref, os_ref, og_ref, ot_ref):
  x = x_ref[...]
  h3 = x + _mlp(x + p_ref[0] + p_ref[1], conv3_refs)
  gid = lax.broadcasted_iota(jnp.int32, (N, G), 1)
  onehot = (batch_ref[...] == gid).astype(jnp.float32)
  g = lax.dot_general(onehot, h3, (((0,), (0,)), ((), ())),
                      preferred_element_type=jnp.float32)
  oc_ref[...] = _head(g, hc_refs)
  os_ref[...] = _head(g, hs_refs)
  og_ref[...] = _head(g, hg_refs)
  ot_ref[...] = _head(g, ht_refs)


def kernel(x, edge_index, batch, params):
  zeros128 = jnp.zeros((N, D), jnp.float32)
  zeros64 = jnp.zeros((N, H), jnp.float32)

  p1 = _seg_sum_128(x, edge_index, zeros128).reshape(2, N, D)
  h1 = _gin_call(x, p1, params["conv1"], has_residual=False)
  p2 = _seg_sum_64(h1, edge_index, zeros64).reshape(2, N, H)
  h2 = _gin_call(h1, p2, params["conv2"], has_residual=True)
  p3 = _seg_sum_64(h2, edge_index, zeros64).reshape(2, N, H)

  outs = pl.pallas_call(
      _final_body,
      out_shape=(
          jax.ShapeDtypeStruct((G, 16), jnp.float32),
          jax.ShapeDtypeStruct((G, 16), jnp.float32),
          jax.ShapeDtypeStruct((G, 8), jnp.float32),
          jax.ShapeDtypeStruct((G, 32), jnp.float32),
      ),
  )(h2, p3, params["conv3"], batch.reshape(N, 1), params["head_color"],
    params["head_size"], params["head_ground"], params["head_struct"])
  return outs


# speedup vs baseline: 8.7804x; 8.7804x over previous
import functools

import jax
import jax.numpy as jnp
from jax import lax
from jax.experimental import pallas as pl
from jax.experimental.pallas import tpu as pltpu
from jax.experimental.pallas import tpu_sc as plsc

N = 10000
E = 640000
D = 128
H = 64
G = 64

CHUNK = 128
NUM_CHUNKS = E // CHUNK
NC = 2
NS = 16
NW = NC * NS
CHUNKS_PER_TILE = (NUM_CHUNKS + NW - 1) // NW
PAD = 10240
SUB_ROWS = PAD // NS


def _make_seg_sum(width):
  mesh = plsc.VectorSubcoreMesh(core_axis_name="c", subcore_axis_name="s")

  @functools.partial(
      pl.kernel,
      out_type=jax.ShapeDtypeStruct((2 * PAD, width), jnp.float32),
      mesh=mesh,
      compiler_params=pltpu.CompilerParams(use_tc_tiling_on_sc=False),
      scratch_types=[
          pltpu.VMEM((CHUNK,), jnp.int32),
          pltpu.VMEM((CHUNK,), jnp.int32),
          pltpu.VMEM((CHUNK, width), jnp.float32),
          pltpu.VMEM_SHARED((PAD, width), jnp.float32),
          pltpu.SemaphoreType.DMA,
      ],
  )
  def seg_sum(feat_hbm, src_hbm, dst_hbm, zeros_hbm, out_hbm,
              src_v, dst_v, rows_v, acc_sh, sem):
    c = lax.axis_index("c")
    s = lax.axis_index("s")
    wid = s * NC + c

    pltpu.sync_copy(zeros_hbm.at[pl.ds(s * SUB_ROWS, SUB_ROWS)],
                    acc_sh.at[pl.ds(s * SUB_ROWS, SUB_ROWS)])
    plsc.subcore_barrier()

    def body(k, carry):
      chunk = k * NW + wid

      @pl.when(chunk < NUM_CHUNKS)
      def _():
        base = chunk * CHUNK
        pltpu.sync_copy(src_hbm.at[pl.ds(base, CHUNK)], src_v)
        pltpu.sync_copy(dst_hbm.at[pl.ds(base, CHUNK)], dst_v)
        pltpu.async_copy(feat_hbm.at[src_v], rows_v, sem).wait()
        pltpu.sync_copy(rows_v, acc_sh.at[dst_v], add=True)

      return carry

    lax.fori_loop(0, CHUNKS_PER_TILE, body, 0)
    plsc.subcore_barrier()

    pltpu.sync_copy(acc_sh.at[pl.ds(s * SUB_ROWS, SUB_ROWS)],
                    out_hbm.at[pl.ds(c * PAD + s * SUB_ROWS, SUB_ROWS)])

  return seg_sum


_seg_sum_128 = _make_seg_sum(D)
_seg_sum_64 = _make_seg_sum(H)


def _bn(a, gamma, beta, eps=1e-5):
  m = jnp.mean(a, axis=0, keepdims=True)
  v = jnp.mean((a - m) ** 2, axis=0, keepdims=True)
  return gamma * (a - m) / jnp.sqrt(v + eps) + beta


def _mlp(h, p):
  a = jnp.dot(h, p["W1"][...], preferred_element_type=jnp.float32) + p["b1"][...]
  a = jnp.maximum(_bn(a, p["g1"][...], p["be1"][...]), 0.0)
  a = jnp.dot(a, p["W2"][...], preferred_element_type=jnp.float32) + p["b2"][...]
  return jnp.maximum(_bn(a, p["g2"][...], p["be2"][...]), 0.0)


def _gin_body(has_residual, x_ref, p_ref, params_refs, o_ref):
  x = x_ref[...]
  h = x + p_ref[0, :N] + p_ref[1, :N]
  out = _mlp(h, params_refs)
  o_ref[...] = x + out if has_residual else out


def _gin_call(x, p, mlp_params, has_residual):
  body = functools.partial(_gin_body, has_residual)
  return pl.pallas_call(
      body,
      out_shape=jax.ShapeDtypeStruct((N, H), jnp.float32),
  )(x, p, mlp_params)


def _head(g, p):
  t = jnp.maximum(
      jnp.dot(g, p["W1"][...], preferred_element_type=jnp.float32) + p["b1"][...], 0.0)
  z = jnp.dot(t, p["W2"][...], preferred_element_type=jnp.float32) + p["b2"][...]
  n = jnp.sqrt(jnp.sum(z * z, axis=1, keepdims=True))
  return z / jnp.maximum(n, 1e-12)


def _final_body(x_ref, p_ref, conv3_refs, batch_ref, hc_refs, hs_refs, hg_refs,
                ht_refs, oc_ref, os_ref, og_ref, ot_ref):
  x = x_ref[...]
  h3 = x + _mlp(x + p_ref[0, :N] + p_ref[1, :N], conv3_refs)
  gid = lax.broadcasted_iota(jnp.int32, (N, G), 1)
  onehot = (batch_ref[...] == gid).astype(jnp.float32)
  g = lax.dot_general(onehot, h3, (((0,), (0,)), ((), ())),
                      preferred_element_type=jnp.float32)
  oc_ref[...] = _head(g, hc_refs)
  os_ref[...] = _head(g, hs_refs)
  og_ref[...] = _head(g, hg_refs)
  ot_ref[...] = _head(g, ht_refs)


def kernel(x, edge_index, batch, params):
  src = edge_index[0]
  dst = edge_index[1]
  zeros128 = jnp.zeros((PAD, D), jnp.float32)
  zeros64 = jnp.zeros((PAD, H), jnp.float32)

  p1 = _seg_sum_128(x, src, dst, zeros128).reshape(2, PAD, D)
  h1 = _gin_call(x, p1, params["conv1"], has_residual=False)
  p2 = _seg_sum_64(h1, src, dst, zeros64).reshape(2, PAD, H)
  h2 = _gin_call(h1, p2, params["conv2"], has_residual=True)
  p3 = _seg_sum_64(h2, src, dst, zeros64).reshape(2, PAD, H)

  outs = pl.pallas_call(
      _final_body,
      out_shape=(
          jax.ShapeDtypeStruct((G, 16), jnp.float32),
          jax.ShapeDtypeStruct((G, 16), jnp.float32),
          jax.ShapeDtypeStruct((G, 8), jnp.float32),
          jax.ShapeDtypeStruct((G, 32), jnp.float32),
      ),
  )(h2, p3, params["conv3"], batch.reshape(N, 1), params["head_color"],
    params["head_size"], params["head_ground"], params["head_struct"])
  return outs


# speedup vs baseline: 23.1934x; 2.6415x over previous
import functools

import jax
import jax.numpy as jnp
from jax import lax
from jax.experimental import pallas as pl
from jax.experimental.pallas import tpu as pltpu
from jax.experimental.pallas import tpu_sc as plsc

N = 10000
E = 640000
D = 128
H = 64
G = 64

CHUNK = 128
NUM_CHUNKS = E // CHUNK
NC = 2
NS = 16
NW = NC * NS
BLK = 4
NUM_BLOCKS = NUM_CHUNKS // BLK
BLOCKS_MAIN = NUM_BLOCKS // NW
UNROLL = 12
OUTER = BLOCKS_MAIN * BLK // UNROLL
PAD = 10240
SUB_ROWS = PAD // NS


def _make_seg_sum(width):
  mesh = plsc.VectorSubcoreMesh(core_axis_name="c", subcore_axis_name="s")

  @functools.partial(
      pl.kernel,
      out_type=jax.ShapeDtypeStruct((2 * PAD, width), jnp.float32),
      mesh=mesh,
      compiler_params=pltpu.CompilerParams(use_tc_tiling_on_sc=False),
      scratch_types=[
          pltpu.VMEM((3, 2, BLK, CHUNK), jnp.int32),
          pltpu.VMEM((2, CHUNK, width), jnp.float32),
          pltpu.VMEM_SHARED((PAD, width), jnp.float32),
          pltpu.SemaphoreType.DMA,
          pltpu.SemaphoreType.DMA,
          pltpu.SemaphoreType.DMA,
          pltpu.SemaphoreType.DMA,
          pltpu.SemaphoreType.DMA,
      ],
  )
  def seg_sum(feat_hbm, ei_hbm, zeros_hbm, out_hbm,
              eblk, rows, acc_sh, bsem0, bsem1, bsem2, gsem0, gsem1):
    bsem = [bsem0, bsem1, bsem2]
    gsem = [gsem0, gsem1]
    c = lax.axis_index("c")
    s = lax.axis_index("s")
    wid = s * NC + c

    def blk_src(g):
      return ei_hbm.at[:, pl.ds((g * NW + wid) * BLK, BLK), :]

    def gather_desc(B, p, b):
      return pltpu.make_async_copy(feat_hbm.at[eblk.at[B, 0, p]],
                                   rows.at[b], gsem[b])

    def scatter(B, p, b):
      pltpu.sync_copy(rows.at[b], acc_sh.at[eblk.at[B, 1, p]], add=True)

    pltpu.sync_copy(zeros_hbm.at[pl.ds(s * SUB_ROWS, SUB_ROWS)],
                    acc_sh.at[pl.ds(s * SUB_ROWS, SUB_ROWS)])
    pltpu.async_copy(blk_src(0), eblk.at[0], bsem[0])
    pltpu.async_copy(blk_src(1), eblk.at[1], bsem[1])
    plsc.subcore_barrier()

    def outer(t, carry):
      for u in range(UNROLL):
        g = t * 3 + u // 4
        B = (u // 4) % 3
        p = u % 4
        b = u % 2
        if p == 0:
          pltpu.make_async_copy(blk_src(g), eblk.at[B], bsem[B]).wait()
        gather_desc(B, p, b).start()
        if u == 0:
          @pl.when(t > 0)
          def _():
            gather_desc(2, 3, 1).wait()
            scatter(2, 3, 1)
        else:
          gather_desc(((u - 1) // 4) % 3, (u - 1) % 4, (u - 1) % 2).wait()
          scatter(((u - 1) // 4) % 3, (u - 1) % 4, (u - 1) % 2)
        if p == 0:
          @pl.when(g + 2 < BLOCKS_MAIN)
          def _():
            pltpu.async_copy(blk_src(g + 2), eblk.at[(B + 2) % 3],
                             bsem[(B + 2) % 3])
      return carry

    lax.fori_loop(0, OUTER, outer, 0)
    gather_desc(2, 3, 1).wait()
    scatter(2, 3, 1)

    @pl.when(wid < NUM_BLOCKS - BLOCKS_MAIN * NW)
    def _():
      pltpu.sync_copy(blk_src(BLOCKS_MAIN), eblk.at[0])
      for p in range(BLK):
        gather_desc(0, p, 0).start()
        gather_desc(0, p, 0).wait()
        scatter(0, p, 0)

    plsc.subcore_barrier()
    pltpu.sync_copy(acc_sh.at[pl.ds(s * SUB_ROWS, SUB_ROWS)],
                    out_hbm.at[pl.ds(c * PAD + s * SUB_ROWS, SUB_ROWS)])

  return seg_sum


_seg_sum_64 = _make_seg_sum(H)


def _bn(a, gamma, beta, eps=1e-5):
  m = jnp.mean(a, axis=0, keepdims=True)
  v = jnp.mean((a - m) ** 2, axis=0, keepdims=True)
  return gamma * (a - m) / jnp.sqrt(v + eps) + beta


def _mlp(h, p):
  a = jnp.dot(h, p["W1"][...], preferred_element_type=jnp.float32) + p["b1"][...]
  a = jnp.maximum(_bn(a, p["g1"][...], p["be1"][...]), 0.0)
  return _mlp_tail(a, p)


def _mlp_tail(a, p):
  a = jnp.dot(a, p["W2"][...], preferred_element_type=jnp.float32) + p["b2"][...]
  return jnp.maximum(_bn(a, p["g2"][...], p["be2"][...]), 0.0)


def _premul_body(x_ref, w_ref, o_ref):
  o_ref[...] = jnp.dot(x_ref[...], w_ref[...], preferred_element_type=jnp.float32)


def _gin1_body(y_ref, p_ref, pr, o_ref):
  a = y_ref[...] + p_ref[0, :N] + p_ref[1, :N] + pr["b1"][...]
  a = jnp.maximum(_bn(a, pr["g1"][...], pr["be1"][...]), 0.0)
  o_ref[...] = _mlp_tail(a, pr)


def _gin_body(x_ref, p_ref, pr, o_ref):
  x = x_ref[...]
  o_ref[...] = x + _mlp(x + p_ref[0, :N] + p_ref[1, :N], pr)


def _head(g, p):
  t = jnp.maximum(
      jnp.dot(g, p["W1"][...], preferred_element_type=jnp.float32) + p["b1"][...], 0.0)
  z = jnp.dot(t, p["W2"][...], preferred_element_type=jnp.float32) + p["b2"][...]
  n = jnp.sqrt(jnp.sum(z * z, axis=1, keepdims=True))
  return z / jnp.maximum(n, 1e-12)


def _final_body(x_ref, p_ref, conv3_refs, batch_ref, hc_refs, hs_refs, hg_refs,
                ht_refs, oc_ref, os_ref, og_ref, ot_ref):
  x = x_ref[...]
  h3 = x + _mlp(x + p_ref[0, :N] + p_ref[1, :N], conv3_refs)
  gid = lax.broadcasted_iota(jnp.int32, (N, G), 1)
  onehot = (batch_ref[...] == gid).astype(jnp.float32)
  g = lax.dot_general(onehot, h3, (((0,), (0,)), ((), ())),
                      preferred_element_type=jnp.float32)
  oc_ref[...] = _head(g, hc_refs)
  os_ref[...] = _head(g, hs_refs)
  og_ref[...] = _head(g, hg_refs)
  ot_ref[...] = _head(g, ht_refs)


def _nh(shape):
  return jax.ShapeDtypeStruct(shape, jnp.float32)


def kernel(x, edge_index, batch, params):
  ei = edge_index.reshape(2, NUM_CHUNKS, CHUNK)
  zeros64 = jnp.zeros((PAD, H), jnp.float32)

  y = pl.pallas_call(_premul_body, out_shape=_nh((N, H)))(x, params["conv1"]["W1"])
  p1 = _seg_sum_64(y, ei, zeros64).reshape(2, PAD, H)
  h1 = pl.pallas_call(_gin1_body, out_shape=_nh((N, H)))(y, p1, params["conv1"])
  p2 = _seg_sum_64(h1, ei, zeros64).reshape(2, PAD, H)
  h2 = pl.pallas_call(_gin_body, out_shape=_nh((N, H)))(h1, p2, params["conv2"])
  p3 = _seg_sum_64(h2, ei, zeros64).reshape(2, PAD, H)

  outs = pl.pallas_call(
      _final_body,
      out_shape=(_nh((G, 16)), _nh((G, 16)), _nh((G, 8)), _nh((G, 32))),
  )(h2, p3, params["conv3"], batch.reshape(N, 1), params["head_color"],
    params["head_size"], params["head_ground"], params["head_struct"])
  return outs


# speedup vs baseline: 27.0354x; 1.1657x over previous
import functools

import jax
import jax.numpy as jnp
from jax import lax
from jax.experimental import pallas as pl
from jax.experimental.pallas import tpu as pltpu
from jax.experimental.pallas import tpu_sc as plsc

N = 10000
E = 640000
D = 128
H = 64
G = 64

CHUNK = 128
NUM_CHUNKS = E // CHUNK
NC = 2
NS = 16
NW = NC * NS
BLK = 4
NUM_BLOCKS = NUM_CHUNKS // BLK
BLOCKS_MAIN = NUM_BLOCKS // NW
UNROLL = 12
OUTER = BLOCKS_MAIN * BLK // UNROLL
PAD = 10240
SUB_ROWS = PAD // NS


def _make_seg_sum(width):
  mesh = plsc.VectorSubcoreMesh(core_axis_name="c", subcore_axis_name="s")

  @functools.partial(
      pl.kernel,
      out_type=jax.ShapeDtypeStruct((2 * PAD, width), jnp.float32),
      mesh=mesh,
      compiler_params=pltpu.CompilerParams(use_tc_tiling_on_sc=False),
      scratch_types=[
          pltpu.VMEM((3, 2, BLK, CHUNK), jnp.int32),
          pltpu.VMEM((4, CHUNK, width), jnp.float32),
          pltpu.VMEM_SHARED((PAD, width), jnp.float32),
          pltpu.SemaphoreType.DMA,
          pltpu.SemaphoreType.DMA,
          pltpu.SemaphoreType.DMA,
          pltpu.SemaphoreType.DMA,
          pltpu.SemaphoreType.DMA,
          pltpu.SemaphoreType.DMA,
          pltpu.SemaphoreType.DMA,
          pltpu.SemaphoreType.DMA,
          pltpu.SemaphoreType.DMA,
          pltpu.SemaphoreType.DMA,
          pltpu.SemaphoreType.DMA,
      ],
  )
  def seg_sum(feat_hbm, ei_hbm, zeros_hbm, out_hbm,
              eblk, rows, acc_sh, bsem0, bsem1, bsem2,
              gsem0, gsem1, gsem2, gsem3, ssem0, ssem1, ssem2, ssem3):
    bsem = [bsem0, bsem1, bsem2]
    gsem = [gsem0, gsem1, gsem2, gsem3]
    ssem = [ssem0, ssem1, ssem2, ssem3]
    c = lax.axis_index("c")
    s = lax.axis_index("s")
    wid = s * NC + c

    def blk_src(g):
      return ei_hbm.at[:, pl.ds((g * NW + wid) * BLK, BLK), :]

    def gather_desc(B, p):
      return pltpu.make_async_copy(feat_hbm.at[eblk.at[B, 0, p]],
                                   rows.at[p], gsem[p])

    def scatter_desc(B, p):
      return pltpu.make_async_copy(rows.at[p], acc_sh.at[eblk.at[B, 1, p]],
                                   ssem[p])

    def scatter_start(B, p):
      pltpu.async_copy(rows.at[p], acc_sh.at[eblk.at[B, 1, p]], ssem[p],
                       add=True)

    pltpu.sync_copy(zeros_hbm.at[pl.ds(s * SUB_ROWS, SUB_ROWS)],
                    acc_sh.at[pl.ds(s * SUB_ROWS, SUB_ROWS)])
    pltpu.async_copy(blk_src(0), eblk.at[0], bsem[0])
    pltpu.async_copy(blk_src(1), eblk.at[1], bsem[1])
    plsc.subcore_barrier()

    def outer(t, carry):
      for u in range(UNROLL):
        k = t * UNROLL + u
        g = t * 3 + u // BLK
        B = (u // BLK) % 3
        p = u % BLK
        if p == 0:
          pltpu.make_async_copy(blk_src(g), eblk.at[B], bsem[B]).wait()

        @pl.when(k >= 4)
        def _():
          scatter_desc(((u + 8) // BLK) % 3, p).wait()

        gather_desc(B, p).start()
        pB, pp = ((u + 10) // BLK) % 3, (u + 2) % 4

        @pl.when(k >= 2)
        def _():
          gather_desc(pB, pp).wait()
          scatter_start(pB, pp)

        if p == 3:
          @pl.when(g + 2 < BLOCKS_MAIN)
          def _():
            nB = (u // BLK + 2) % 3
            pltpu.async_copy(blk_src(g + 2), eblk.at[nB], bsem[nB])
      return carry

    lax.fori_loop(0, OUTER, outer, 0)
    gather_desc(2, 2).wait()
    scatter_start(2, 2)
    gather_desc(2, 3).wait()
    scatter_start(2, 3)
    scatter_desc(2, 0).wait()
    scatter_desc(2, 1).wait()
    scatter_desc(2, 2).wait()
    scatter_desc(2, 3).wait()

    @pl.when(wid < NUM_BLOCKS - BLOCKS_MAIN * NW)
    def _():
      pltpu.sync_copy(blk_src(BLOCKS_MAIN), eblk.at[0])
      for p in range(BLK):
        gather_desc(0, p).start()
        gather_desc(0, p).wait()
        pltpu.sync_copy(rows.at[p], acc_sh.at[eblk.at[0, 1, p]], add=True)

    plsc.subcore_barrier()
    pltpu.sync_copy(acc_sh.at[pl.ds(s * SUB_ROWS, SUB_ROWS)],
                    out_hbm.at[pl.ds(c * PAD + s * SUB_ROWS, SUB_ROWS)])

  return seg_sum


_seg_sum_64 = _make_seg_sum(H)


def _bn(a, gamma, beta, eps=1e-5):
  m = jnp.mean(a, axis=0, keepdims=True)
  v = jnp.mean((a - m) ** 2, axis=0, keepdims=True)
  return gamma * (a - m) / jnp.sqrt(v + eps) + beta


def _mlp(h, p):
  a = jnp.dot(h, p["W1"][...], preferred_element_type=jnp.float32) + p["b1"][...]
  a = jnp.maximum(_bn(a, p["g1"][...], p["be1"][...]), 0.0)
  return _mlp_tail(a, p)


def _mlp_tail(a, p):
  a = jnp.dot(a, p["W2"][...], preferred_element_type=jnp.float32) + p["b2"][...]
  return jnp.maximum(_bn(a, p["g2"][...], p["be2"][...]), 0.0)


def _premul_body(x_ref, w_ref, o_ref):
  o_ref[...] = jnp.dot(x_ref[...], w_ref[...], preferred_element_type=jnp.float32)


def _gin1_body(y_ref, p_ref, pr, o_ref):
  a = y_ref[...] + p_ref[0, :N] + p_ref[1, :N] + pr["b1"][...]
  a = jnp.maximum(_bn(a, pr["g1"][...], pr["be1"][...]), 0.0)
  o_ref[...] = _mlp_tail(a, pr)


def _gin_body(x_ref, p_ref, pr, o_ref):
  x = x_ref[...]
  o_ref[...] = x + _mlp(x + p_ref[0, :N] + p_ref[1, :N], pr)


def _head(g, p):
  t = jnp.maximum(
      jnp.dot(g, p["W1"][...], preferred_element_type=jnp.float32) + p["b1"][...], 0.0)
  z = jnp.dot(t, p["W2"][...], preferred_element_type=jnp.float32) + p["b2"][...]
  n = jnp.sqrt(jnp.sum(z * z, axis=1, keepdims=True))
  return z / jnp.maximum(n, 1e-12)


def _final_body(x_ref, p_ref, conv3_refs, batch_ref, hc_refs, hs_refs, hg_refs,
                ht_refs, oc_ref, os_ref, og_ref, ot_ref):
  x = x_ref[...]
  h3 = x + _mlp(x + p_ref[0, :N] + p_ref[1, :N], conv3_refs)
  gid = lax.broadcasted_iota(jnp.int32, (N, G), 1)
  onehot = (batch_ref[...] == gid).astype(jnp.float32)
  g = lax.dot_general(onehot, h3, (((0,), (0,)), ((), ())),
                      preferred_element_type=jnp.float32)
  oc_ref[...] = _head(g, hc_refs)
  os_ref[...] = _head(g, hs_refs)
  og_ref[...] = _head(g, hg_refs)
  ot_ref[...] = _head(g, ht_refs)


def _nh(shape):
  return jax.ShapeDtypeStruct(shape, jnp.float32)


def kernel(x, edge_index, batch, params):
  ei = edge_index.reshape(2, NUM_CHUNKS, CHUNK)
  zeros64 = jnp.zeros((PAD, H), jnp.float32)

  y = pl.pallas_call(_premul_body, out_shape=_nh((N, H)))(x, params["conv1"]["W1"])
  p1 = _seg_sum_64(y, ei, zeros64).reshape(2, PAD, H)
  h1 = pl.pallas_call(_gin1_body, out_shape=_nh((N, H)))(y, p1, params["conv1"])
  p2 = _seg_sum_64(h1, ei, zeros64).reshape(2, PAD, H)
  h2 = pl.pallas_call(_gin_body, out_shape=_nh((N, H)))(h1, p2, params["conv2"])
  p3 = _seg_sum_64(h2, ei, zeros64).reshape(2, PAD, H)

  outs = pl.pallas_call(
      _final_body,
      out_shape=(_nh((G, 16)), _nh((G, 16)), _nh((G, 8)), _nh((G, 32))),
  )(h2, p3, params["conv3"], batch.reshape(N, 1), params["head_color"],
    params["head_size"], params["head_ground"], params["head_struct"])
  return outs
